# baseline (speedup 1.0000x reference)
import functools

import jax
import jax.numpy as jnp
from jax import lax
from jax.experimental import pallas as pl
from jax.experimental.pallas import tpu as pltpu
from jax.experimental.pallas import tpu_sc as plsc

B = 16384
W = 1000
WP = 1024
NS = 16
CHUNK = B // NS
ROWS = CHUNK // 128
L = 16


def _sc_loss(inp_hbm, tgt_hbm, idx_hbm, hneg_hbm, zero_hbm, out_hbm,
             tgt_v, idx_v, picked_v, ones_v,
             hloc_v, sloc_v, nloc_v, out_v,
             h_s, s_s, sem_g, sem_a):
    wid = lax.axis_index("s")
    fzero = jnp.zeros((L,), jnp.float32)
    fone = jnp.ones((L,), jnp.float32)

    @pl.when(wid == 0)
    def _zero_shared():
        pltpu.sync_copy(zero_hbm, h_s)
        pltpu.sync_copy(zero_hbm, s_s)
        pltpu.sync_copy(hneg_hbm, nloc_v)

    pltpu.sync_copy(tgt_hbm.at[wid], tgt_v)
    pltpu.sync_copy(idx_hbm.at[wid], idx_v)
    for j in range(128 // L):
        ones_v[pl.ds(j * L, L)] = fone

    gathers = [
        pltpu.async_copy(inp_hbm.at[idx_v.at[r]], picked_v.at[r],
                         sem_g.at[r])
        for r in range(ROWS)
    ]

    plsc.subcore_barrier()

    adds = [
        pltpu.async_copy(ones_v, h_s.at[tgt_v.at[r]], sem_a, add=True)
        for r in range(ROWS)
    ]
    for r in range(ROWS):
        gathers[r].wait()
        adds.append(pltpu.async_copy(picked_v.at[r], s_s.at[tgt_v.at[r]],
                                     sem_a, add=True))
    for a in adds:
        a.wait()

    plsc.subcore_barrier()

    @pl.when(wid == 0)
    def _epilogue():
        cp_h = pltpu.async_copy(h_s, hloc_v, sem_g.at[0])
        cp_s = pltpu.async_copy(s_s, sloc_v, sem_a)
        cp_h.wait()
        cp_s.wait()

        def body(j, acc):
            num_acc, den_acc = acc
            hh = hloc_v[pl.ds(j * L, L)]
            ss = sloc_v[pl.ds(j * L, L)]
            cc = hh + nloc_v[pl.ds(j * L, L)]
            return (num_acc + cc * ss, den_acc + cc * hh)

        num_acc, den_acc = lax.fori_loop(0, WP // L, body, (fzero, fzero))
        num = jnp.broadcast_to(jnp.sum(num_acc), (L,))
        den = jnp.broadcast_to(jnp.sum(den_acc), (L,))
        out_v[...] = -(num / den)
        pltpu.sync_copy(out_v, out_hbm)


@functools.partial(
    pl.kernel,
    out_type=jax.ShapeDtypeStruct((L,), jnp.float32),
    mesh=plsc.VectorSubcoreMesh(core_axis_name="c", subcore_axis_name="s",
                                num_cores=1),
    compiler_params=pltpu.CompilerParams(needs_layout_passes=False),
    scratch_types=[
        pltpu.VMEM((ROWS, 128), jnp.int32),
        pltpu.VMEM((ROWS, 128), jnp.int32),
        pltpu.VMEM((ROWS, 128), jnp.float32),
        pltpu.VMEM((128,), jnp.float32),
        pltpu.VMEM((WP,), jnp.float32),
        pltpu.VMEM((WP,), jnp.float32),
        pltpu.VMEM((WP,), jnp.float32),
        pltpu.VMEM((L,), jnp.float32),
        pltpu.VMEM_SHARED((WP,), jnp.float32),
        pltpu.VMEM_SHARED((WP,), jnp.float32),
        pltpu.SemaphoreType.DMA((ROWS,)),
        pltpu.SemaphoreType.DMA,
    ],
)
def _negloss_sc(inp_hbm, tgt_hbm, idx_hbm, hneg_hbm, zero_hbm, out_hbm,
                *scratch):
    _sc_loss(inp_hbm, tgt_hbm, idx_hbm, hneg_hbm, zero_hbm, out_hbm, *scratch)


def kernel(input, target, distr):
    num_words = distr.shape[0]
    neg = jax.random.categorical(jax.random.key(42), jnp.log(distr),
                                 shape=(5,))
    hneg = jnp.zeros((WP,), jnp.float32).at[neg].add(1.0)
    inp_flat = input.reshape(128, 128, 125, 8).transpose(2, 0, 3, 1).reshape(-1)
    t = target.astype(jnp.int32)
    i = jnp.arange(B, dtype=jnp.int32)
    idx = ((t >> 3) * (128 * 8 * 128) + (i >> 7) * (8 * 128)
           + (t & 7) * 128 + (i & 127))
    tgt3 = t.reshape(NS, ROWS, 128)
    idx3 = idx.reshape(NS, ROWS, 128)
    zero = jnp.zeros((WP,), jnp.float32)
    out = _negloss_sc(inp_flat, tgt3, idx3, hneg, zero)
    del num_words
    return out[0]


# speedup vs baseline: 1.0784x; 1.0784x over previous
import functools

import jax
import jax.numpy as jnp
from jax import lax
from jax.experimental import pallas as pl
from jax.experimental.pallas import tpu as pltpu
from jax.experimental.pallas import tpu_sc as plsc

B = 16384
W = 1000
WP = 1024
NS = 16
CHUNK = B // NS
ROWS = CHUNK // 128
L = 16


def _sc_loss(inp_hbm, tgt_hbm, hneg_hbm, zero_hbm, out_hbm,
             tgt_v, idx_v, picked_v, ones_v,
             hloc_v, sloc_v, nloc_v, out_v,
             h_s, s_s, sem_g, sem_a):
    wid = lax.axis_index("s")
    fzero = jnp.zeros((L,), jnp.float32)
    fone = jnp.ones((L,), jnp.float32)

    @pl.when(wid == 0)
    def _zero_shared():
        pltpu.sync_copy(zero_hbm, h_s)
        pltpu.sync_copy(zero_hbm, s_s)
        pltpu.sync_copy(hneg_hbm, nloc_v)

    pltpu.sync_copy(tgt_hbm.at[wid], tgt_v)
    iota = lax.iota(jnp.int32, L)
    for j in range(128 // L):
        ones_v[pl.ds(j * L, L)] = fone
    for r in range(ROWS):
        for c in range(128 // L):
            t16 = tgt_v[r, pl.ds(c * L, L)]
            row = wid * CHUNK + r * 128 + c * L + iota
            idx_v[r, pl.ds(c * L, L)] = (
                (t16 >> 3) * (128 * 8 * 128)
                + (row >> 7) * (8 * 128)
                + (t16 & 7) * 128
                + (row & 127)
            )

    gathers = [
        pltpu.async_copy(inp_hbm.at[idx_v.at[r]], picked_v.at[r],
                         sem_g.at[r])
        for r in range(ROWS)
    ]

    plsc.subcore_barrier()

    adds = [
        pltpu.async_copy(ones_v, h_s.at[tgt_v.at[r]], sem_a, add=True)
        for r in range(ROWS)
    ]
    for r in range(ROWS):
        gathers[r].wait()
        adds.append(pltpu.async_copy(picked_v.at[r], s_s.at[tgt_v.at[r]],
                                     sem_a, add=True))
    for a in adds:
        a.wait()

    plsc.subcore_barrier()

    @pl.when(wid == 0)
    def _epilogue():
        cp_h = pltpu.async_copy(h_s, hloc_v, sem_g.at[0])
        cp_s = pltpu.async_copy(s_s, sloc_v, sem_a)
        cp_h.wait()
        cp_s.wait()

        def body(j, acc):
            num_acc, den_acc = acc
            hh = hloc_v[pl.ds(j * L, L)]
            ss = sloc_v[pl.ds(j * L, L)]
            cc = hh + nloc_v[pl.ds(j * L, L)]
            return (num_acc + cc * ss, den_acc + cc * hh)

        num_acc, den_acc = lax.fori_loop(0, WP // L, body, (fzero, fzero))
        num = jnp.broadcast_to(jnp.sum(num_acc), (L,))
        den = jnp.broadcast_to(jnp.sum(den_acc), (L,))
        out_v[...] = -(num / den)
        pltpu.sync_copy(out_v, out_hbm)


@functools.partial(
    pl.kernel,
    out_type=jax.ShapeDtypeStruct((L,), jnp.float32),
    mesh=plsc.VectorSubcoreMesh(core_axis_name="c", subcore_axis_name="s",
                                num_cores=1),
    compiler_params=pltpu.CompilerParams(needs_layout_passes=False),
    scratch_types=[
        pltpu.VMEM((ROWS, 128), jnp.int32),
        pltpu.VMEM((ROWS, 128), jnp.int32),
        pltpu.VMEM((ROWS, 128), jnp.float32),
        pltpu.VMEM((128,), jnp.float32),
        pltpu.VMEM((WP,), jnp.float32),
        pltpu.VMEM((WP,), jnp.float32),
        pltpu.VMEM((WP,), jnp.float32),
        pltpu.VMEM((L,), jnp.float32),
        pltpu.VMEM_SHARED((WP,), jnp.float32),
        pltpu.VMEM_SHARED((WP,), jnp.float32),
        pltpu.SemaphoreType.DMA((ROWS,)),
        pltpu.SemaphoreType.DMA,
    ],
)
def _negloss_sc(inp_hbm, tgt_hbm, hneg_hbm, zero_hbm, out_hbm, *scratch):
    _sc_loss(inp_hbm, tgt_hbm, hneg_hbm, zero_hbm, out_hbm, *scratch)


def kernel(input, target, distr):
    num_words = distr.shape[0]
    neg = jax.random.categorical(jax.random.key(42), jnp.log(distr),
                                 shape=(5,))
    hneg = jnp.zeros((WP,), jnp.float32).at[neg].add(1.0)
    inp_flat = input.reshape(128, 128, 125, 8).transpose(2, 0, 3, 1).reshape(-1)
    tgt3 = target.astype(jnp.int32).reshape(NS, ROWS, 128)
    zero = jnp.zeros((WP,), jnp.float32)
    out = _negloss_sc(inp_flat, tgt3, hneg, zero)
    del num_words
    return out[0]


# speedup vs baseline: 1.0794x; 1.0009x over previous
import functools

import jax
import jax.numpy as jnp
from jax import lax
from jax.experimental import pallas as pl
from jax.experimental.pallas import tpu as pltpu
from jax.experimental.pallas import tpu_sc as plsc

B = 16384
W = 1000
WP = 1024
NS = 16
CHUNK = B // NS
ROWS = CHUNK // 128
L = 16


def _sc_loss(inp_hbm, tgt_hbm, hneg_hbm, zero_hbm, out_hbm,
             tgt_v, idx_v, picked_v, ones_v,
             hloc_v, sloc_v, nloc_v, out_v,
             h_s, s_s, sem_g, sem_a):
    wid = lax.axis_index("s")
    fzero = jnp.zeros((L,), jnp.float32)
    fone = jnp.ones((L,), jnp.float32)

    @pl.when(wid == 0)
    def _zero_shared():
        pltpu.sync_copy(zero_hbm, h_s)
        pltpu.sync_copy(zero_hbm, s_s)
        pltpu.sync_copy(hneg_hbm, nloc_v)

    pltpu.sync_copy(tgt_hbm.at[wid], tgt_v)
    iota = lax.iota(jnp.int32, L)
    for j in range(128 // L):
        ones_v[pl.ds(j * L, L)] = fone
    for r in range(ROWS):
        for c in range(128 // L):
            t16 = tgt_v[r, pl.ds(c * L, L)]
            row = wid * CHUNK + r * 128 + c * L + iota
            idx_v[r, pl.ds(c * L, L)] = (
                (t16 >> 3) * (128 * 8 * 128)
                + (row >> 7) * (8 * 128)
                + (t16 & 7) * 128
                + (row & 127)
            )

    gathers = [
        pltpu.async_copy(inp_hbm.at[idx_v.at[r]], picked_v.at[r],
                         sem_g.at[r])
        for r in range(ROWS)
    ]

    plsc.subcore_barrier()

    adds = [
        pltpu.async_copy(ones_v, h_s.at[tgt_v.at[r]], sem_a, add=True)
        for r in range(ROWS)
    ]
    for r in range(ROWS):
        gathers[r].wait()
        adds.append(pltpu.async_copy(picked_v.at[r], s_s.at[tgt_v.at[r]],
                                     sem_a, add=True))
    for a in adds:
        a.wait()

    plsc.subcore_barrier()

    @pl.when(wid == 0)
    def _epilogue():
        cp_h = pltpu.async_copy(h_s, hloc_v, sem_g.at[0])
        cp_s = pltpu.async_copy(s_s, sloc_v, sem_a)
        cp_h.wait()
        cp_s.wait()

        def body(j, acc):
            num_acc, den_acc = acc
            hh = hloc_v[pl.ds(j * L, L)]
            ss = sloc_v[pl.ds(j * L, L)]
            cc = hh + nloc_v[pl.ds(j * L, L)]
            return (num_acc + cc * ss, den_acc + cc * hh)

        num_acc, den_acc = lax.fori_loop(0, WP // L, body, (fzero, fzero))
        num = jnp.broadcast_to(jnp.sum(num_acc), (L,))
        den = jnp.broadcast_to(jnp.sum(den_acc), (L,))
        out_v[...] = -(num / den)
        pltpu.sync_copy(out_v, out_hbm)


@functools.partial(
    pl.kernel,
    out_type=jax.ShapeDtypeStruct((L,), jnp.float32),
    mesh=plsc.VectorSubcoreMesh(core_axis_name="c", subcore_axis_name="s",
                                num_cores=1),
    compiler_params=pltpu.CompilerParams(needs_layout_passes=False,
                                         skip_device_barrier=True),
    scratch_types=[
        pltpu.VMEM((ROWS, 128), jnp.int32),
        pltpu.VMEM((ROWS, 128), jnp.int32),
        pltpu.VMEM((ROWS, 128), jnp.float32),
        pltpu.VMEM((128,), jnp.float32),
        pltpu.VMEM((WP,), jnp.float32),
        pltpu.VMEM((WP,), jnp.float32),
        pltpu.VMEM((WP,), jnp.float32),
        pltpu.VMEM((L,), jnp.float32),
        pltpu.VMEM_SHARED((WP,), jnp.float32),
        pltpu.VMEM_SHARED((WP,), jnp.float32),
        pltpu.SemaphoreType.DMA((ROWS,)),
        pltpu.SemaphoreType.DMA,
    ],
)
def _negloss_sc(inp_hbm, tgt_hbm, hneg_hbm, zero_hbm, out_hbm, *scratch):
    _sc_loss(inp_hbm, tgt_hbm, hneg_hbm, zero_hbm, out_hbm, *scratch)


def kernel(input, target, distr):
    num_words = distr.shape[0]
    neg = jax.random.categorical(jax.random.key(42), jnp.log(distr),
                                 shape=(5,))
    hneg = jnp.zeros((WP,), jnp.float32).at[neg].add(1.0)
    inp_flat = input.reshape(128, 128, 125, 8).transpose(2, 0, 3, 1).reshape(-1)
    tgt3 = target.astype(jnp.int32).reshape(NS, ROWS, 128)
    zero = jnp.zeros((WP,), jnp.float32)
    out = _negloss_sc(inp_flat, tgt3, hneg, zero)
    del num_words
    return out[0]


# speedup vs baseline: 1.1165x; 1.0344x over previous
import functools

import jax
import jax.numpy as jnp
from jax import lax
from jax.experimental import pallas as pl
from jax.experimental.pallas import tpu as pltpu
from jax.experimental.pallas import tpu_sc as plsc

B = 16384
W = 1000
WP = 1024
NS = 16
CHUNK = B // NS
ROWS = CHUNK // 128
L = 16


def _sc_loss(inp_hbm, tgt_hbm, hneg_hbm, out_hbm,
             tgt_v, idx_v, picked_v, ones_v,
             zeros_v, hloc_v, sloc_v, nloc_v, out_v,
             h_s, s_s, sem_g, sem_a):
    wid = lax.axis_index("s")
    iota = lax.iota(jnp.int32, L)
    fzero = jnp.zeros((L,), jnp.float32)
    fone = jnp.ones((L,), jnp.float32)

    @pl.when(wid == 0)
    def _zero_shared():
        for j in range(WP // L):
            zeros_v[pl.ds(j * L, L)] = fzero
        pltpu.sync_copy(zeros_v, h_s)
        pltpu.sync_copy(zeros_v, s_s)
        pltpu.sync_copy(hneg_hbm, nloc_v)

    pltpu.sync_copy(tgt_hbm.at[wid], tgt_v)
    for r in range(ROWS):
        for c in range(128 // L):
            t16 = tgt_v[r, pl.ds(c * L, L)]
            row = wid * CHUNK + r * 128 + c * L + iota
            idx_v[r, pl.ds(c * L, L)] = (
                (t16 >> 3) * (128 * 8 * 128)
                + (row >> 7) * (8 * 128)
                + (t16 & 7) * 128
                + (row & 127)
            )
            ones_v[r, pl.ds(c * L, L)] = fone

    gathers = [
        pltpu.async_copy(inp_hbm.at[idx_v.at[r]], picked_v.at[r],
                         sem_g.at[r])
        for r in range(ROWS)
    ]

    plsc.subcore_barrier()

    adds = [
        pltpu.async_copy(ones_v.at[r], h_s.at[tgt_v.at[r]], sem_a, add=True)
        for r in range(ROWS)
    ]
    for r in range(ROWS):
        gathers[r].wait()
        adds.append(pltpu.async_copy(picked_v.at[r], s_s.at[tgt_v.at[r]],
                                     sem_a, add=True))
    for a in adds:
        a.wait()

    plsc.subcore_barrier()

    @pl.when(wid == 0)
    def _epilogue():
        cp_h = pltpu.async_copy(h_s, hloc_v, sem_g.at[0])
        cp_s = pltpu.async_copy(s_s, sloc_v, sem_a)
        cp_h.wait()
        cp_s.wait()
        num_acc = fzero
        den_acc = fzero
        for j in range(WP // L):
            hh = hloc_v[pl.ds(j * L, L)]
            ss = sloc_v[pl.ds(j * L, L)]
            cc = hh + nloc_v[pl.ds(j * L, L)]
            num_acc = num_acc + cc * ss
            den_acc = den_acc + cc * hh
        num = jnp.broadcast_to(jnp.sum(num_acc), (L,))
        den = jnp.broadcast_to(jnp.sum(den_acc), (L,))
        out_v[...] = -(num / den)
        pltpu.sync_copy(out_v, out_hbm)


@functools.partial(
    pl.kernel,
    out_type=jax.ShapeDtypeStruct((L,), jnp.float32),
    mesh=plsc.VectorSubcoreMesh(core_axis_name="c", subcore_axis_name="s",
                                num_cores=1),
    compiler_params=pltpu.CompilerParams(needs_layout_passes=False),
    scratch_types=[
        pltpu.VMEM((ROWS, 128), jnp.int32),
        pltpu.VMEM((ROWS, 128), jnp.int32),
        pltpu.VMEM((ROWS, 128), jnp.float32),
        pltpu.VMEM((ROWS, 128), jnp.float32),
        pltpu.VMEM((WP,), jnp.float32),
        pltpu.VMEM((WP,), jnp.float32),
        pltpu.VMEM((WP,), jnp.float32),
        pltpu.VMEM((WP,), jnp.float32),
        pltpu.VMEM((L,), jnp.float32),
        pltpu.VMEM_SHARED((WP,), jnp.float32),
        pltpu.VMEM_SHARED((WP,), jnp.float32),
        pltpu.SemaphoreType.DMA((ROWS,)),
        pltpu.SemaphoreType.DMA,
    ],
)
def _negloss_sc(inp_hbm, tgt_hbm, hneg_hbm, out_hbm, *scratch):
    _sc_loss(inp_hbm, tgt_hbm, hneg_hbm, out_hbm, *scratch)


def kernel(input, target, distr):
    num_words = distr.shape[0]
    neg = jax.random.categorical(jax.random.key(42), jnp.log(distr),
                                 shape=(5,))
    hneg = jnp.zeros((WP,), jnp.float32).at[neg].add(1.0)
    inp_flat = input.reshape(128, 128, 125, 8).transpose(2, 0, 3, 1).reshape(-1)
    tgt3 = target.astype(jnp.int32).reshape(NS, ROWS, 128)
    out = _negloss_sc(inp_flat, tgt3, hneg)
    del num_words
    return out[0]
